# Initial kernel scaffold; baseline (speedup 1.0000x reference)
import jax
import jax.numpy as jnp
from jax.experimental import pallas as pl


def kernel(input_ids, attention_mask, question_time, embed_table, Wq, Wv, kb_keys, kb_values, kb_ctx, tau_min, tau_max, w_ctx, w_gate):
    raise NotImplementedError("write your pallas kernel here")


# speedup vs baseline: 1.6906x; 1.6906x over previous
import functools

import jax
import jax.numpy as jnp
from jax import lax
from jax.experimental import pallas as pl
from jax.experimental.pallas import tpu as pltpu
from jax.experimental.pallas import tpu_sc as plsc

BB, TT = 8, 4
NQ = BB * TT
D_MODEL = 1024
D_K = 64
D_V = 64
D_CTX = 16
K_TOP = 32
KB_N = 1000000
VOCAB = 32000

CHUNK = 4096
GSIZE = 32
BPC = CHUNK // GSIZE
NSTEP = (KB_N + CHUNK - 1) // CHUNK
NBLK = NSTEP * BPC
NCAND = K_TOP * GSIZE
JBLK = 8
VCHUNK = 1280
NEG = -1e30


def _sc_gather_embed(table, idx, n_workers=4):
    B = idx.shape[0]
    V, D = table.shape
    b_per_w = B // n_workers
    info = plsc.get_sparse_core_info()
    NC = info.num_cores
    mesh = plsc.VectorSubcoreMesh(core_axis_name="c", subcore_axis_name="s")

    @functools.partial(
        pl.kernel, mesh=mesh,
        out_type=jax.ShapeDtypeStruct((B, D), table.dtype),
        scratch_types=[
            pltpu.VMEM((b_per_w,), jnp.int32),
            pltpu.VMEM((b_per_w, D), table.dtype),
            pltpu.SemaphoreType.DMA,
        ],
    )
    def k(table_hbm, idx_hbm, out_hbm, idx_v, rows_v, sem):
        wid = lax.axis_index("s") * NC + lax.axis_index("c")

        @pl.when(wid < n_workers)
        def _():
            base = wid * b_per_w
            pltpu.sync_copy(idx_hbm.at[pl.ds(base, b_per_w)], idx_v)
            pltpu.async_copy(table_hbm.at[idx_v], rows_v, sem).wait()
            pltpu.sync_copy(rows_v, out_hbm.at[pl.ds(base, b_per_w)])

    return k(table, idx)


def _a_body(hid_ref, wq_ref, keys_ref, q_out, bmax_out, q_s):
    i = pl.program_id(0)

    @pl.when(i == 0)
    def _():
        q0 = lax.dot_general(hid_ref[...], wq_ref[...],
                             (((1,), (0,)), ((), ())),
                             preferred_element_type=jnp.float32)
        q_s[...] = q0
        q_out[...] = q0

    q = q_s[...]
    s = lax.dot_general(keys_ref[...], q, (((1,), (1,)), ((), ())),
                        preferred_element_type=jnp.float32)
    s3 = s.reshape(BPC, GSIZE, NQ)
    u3 = lax.broadcasted_iota(jnp.int32, s3.shape, 0)
    v3 = lax.broadcasted_iota(jnp.int32, s3.shape, 1)
    col = i * CHUNK + u3 * GSIZE + v3
    s3 = jnp.where(col < KB_N, s3, NEG)
    bmax_out[...] = s3.max(axis=1)


def _tc_scores_bmax(hidden, Wq, kb_keys, interpret=False):
    return pl.pallas_call(
        _a_body,
        grid=(NSTEP,),
        in_specs=[
            pl.BlockSpec((NQ, D_MODEL), lambda i: (0, 0)),
            pl.BlockSpec((D_MODEL, D_K), lambda i: (0, 0)),
            pl.BlockSpec((CHUNK, D_K), lambda i: (i, 0)),
        ],
        out_specs=[
            pl.BlockSpec((NQ, D_K), lambda i: (0, 0)),
            pl.BlockSpec((BPC, NQ), lambda i: (i, 0)),
        ],
        out_shape=[
            jax.ShapeDtypeStruct((NQ, D_K), jnp.float32),
            jax.ShapeDtypeStruct((NBLK, NQ), jnp.float32),
        ],
        scratch_shapes=[pltpu.VMEM((NQ, D_K), jnp.float32)],
        compiler_params=pltpu.CompilerParams(
            dimension_semantics=("arbitrary",)),
        interpret=interpret,
    )(hidden, Wq, kb_keys)


def _b_body(bmax_ref, bids_out, vals_s):
    vals_s[...] = bmax_ref[...]
    pos = lax.broadcasted_iota(jnp.int32, (NQ, NBLK), 1)
    ko = lax.broadcasted_iota(jnp.int32, (NQ, K_TOP), 1)

    def step(a, acc):
        vals = vals_s[...]
        m = vals.max(axis=1, keepdims=True)
        p = jnp.where(vals == m, pos, jnp.int32(2**30)).min(
            axis=1, keepdims=True)
        acc = jnp.where(ko == a, p, acc)
        vals_s[...] = jnp.where(pos == p, NEG, vals)
        return acc

    bids_out[...] = lax.fori_loop(
        0, K_TOP, step, jnp.zeros((NQ, K_TOP), jnp.int32))


def _tc_select_blocks(bmax, interpret=False):
    return pl.pallas_call(
        _b_body,
        out_shape=jax.ShapeDtypeStruct((NQ, K_TOP), jnp.int32),
        scratch_shapes=[pltpu.VMEM((NQ, NBLK), jnp.float32)],
        interpret=interpret,
    )(bmax)


def _c_body(ids_ref, *refs):
    krefs = refs[0:JBLK]
    vrefs = refs[JBLK:2 * JBLK]
    crefs = refs[2 * JBLK:3 * JBLK]
    trefs = refs[3 * JBLK:4 * JBLK]
    q_ref, qmin_ref, qmax_ref, wctx_ref = refs[4 * JBLK:4 * JBLK + 4]
    s_out, att_out, v_out = refs[4 * JBLK + 4:]

    i = pl.program_id(0)
    r = i // (K_TOP // JBLK)
    rowoh = (lax.broadcasted_iota(jnp.int32, (NQ, 1), 0) == r
             ).astype(jnp.float32)
    qr = (q_ref[...] * rowoh).sum(axis=0, keepdims=True)
    qmn = (qmin_ref[...] * rowoh).sum(axis=0, keepdims=True)
    qmx = (qmax_ref[...] * rowoh).sum(axis=0, keepdims=True)
    io8 = lax.broadcasted_iota(jnp.int32, (8, 1), 0)

    for j in range(JBLK):
        bid = ids_ref[i * JBLK + j]
        s_row = lax.dot_general(qr, krefs[j][...], (((1,), (1,)), ((), ())),
                                preferred_element_type=jnp.float32)
        cb = lax.dot_general(wctx_ref[...], crefs[j][...],
                             (((1,), (1,)), ((), ())),
                             preferred_element_type=jnp.float32)
        toh = (io8 == (bid % 8)).astype(jnp.float32)
        trow = (trefs[j][...] * toh).sum(axis=0, keepdims=True)
        tmin = trow[:, 0:GSIZE]
        tmax = trow[:, GSIZE:2 * GSIZE]
        att = s_row * jnp.float32(0.125) + cb
        valid = (tmin <= qmx) & (tmax >= qmn)
        att = jnp.where(valid, att, jnp.float32(-1e9))
        s_out[pl.ds(j, 1), :] = s_row
        att_out[pl.ds(j, 1), :] = att
        v_out[pl.ds(j, 1), :, :] = vrefs[j][...][None]


def _tc_fetch_score(bids_flat, kb_keys, kb_values, kb_ctx, taupack,
                    q, qmin, qmax, wctx2, interpret=False):
    nsb = NQ * K_TOP
    grid = (nsb // JBLK,)

    def kmap(j):
        return lambda i, ids: (ids[i * JBLK + j], 0)

    def tmap(j):
        return lambda i, ids: (ids[i * JBLK + j] // 8, 0)

    in_specs = (
        [pl.BlockSpec((GSIZE, D_K), kmap(j)) for j in range(JBLK)]
        + [pl.BlockSpec((GSIZE, D_V), kmap(j)) for j in range(JBLK)]
        + [pl.BlockSpec((GSIZE, D_CTX), kmap(j)) for j in range(JBLK)]
        + [pl.BlockSpec((8, 2 * GSIZE), tmap(j)) for j in range(JBLK)]
        + [
            pl.BlockSpec((NQ, D_K), lambda i, ids: (0, 0)),
            pl.BlockSpec((NQ, 1), lambda i, ids: (0, 0)),
            pl.BlockSpec((NQ, 1), lambda i, ids: (0, 0)),
            pl.BlockSpec((1, D_CTX), lambda i, ids: (0, 0)),
        ]
    )
    out_specs = [
        pl.BlockSpec((JBLK, GSIZE), lambda i, ids: (i, 0)),
        pl.BlockSpec((JBLK, GSIZE), lambda i, ids: (i, 0)),
        pl.BlockSpec((JBLK, GSIZE, D_V), lambda i, ids: (i, 0, 0)),
    ]
    return pl.pallas_call(
        _c_body,
        grid_spec=pltpu.PrefetchScalarGridSpec(
            num_scalar_prefetch=1,
            grid=grid,
            in_specs=in_specs,
            out_specs=out_specs,
        ),
        out_shape=[
            jax.ShapeDtypeStruct((nsb, GSIZE), jnp.float32),
            jax.ShapeDtypeStruct((nsb, GSIZE), jnp.float32),
            jax.ShapeDtypeStruct((nsb, GSIZE, D_V), jnp.float32),
        ],
        compiler_params=pltpu.CompilerParams(
            dimension_semantics=("arbitrary",)),
        interpret=interpret,
    )(bids_flat,
      *([kb_keys] * JBLK), *([kb_values] * JBLK), *([kb_ctx] * JBLK),
      *([taupack] * JBLK), q, qmin, qmax, wctx2)


def _d_body(s_ref, att_ref, v3_ref, hid_ref, wv_ref, wgate_ref, amask_ref,
            hnew_out, vals_s):
    vals_s[...] = s_ref[...]
    att_all = att_ref[...]
    pos = lax.broadcasted_iota(jnp.int32, (NQ, NCAND), 1)

    def step(a, _):
        vals = vals_s[...]
        m = vals.max(axis=1, keepdims=True)
        p = jnp.where(vals == m, pos, jnp.int32(2**30)).min(
            axis=1, keepdims=True)
        vals_s[...] = jnp.where(pos == p, NEG, vals)
        return 0

    lax.fori_loop(0, K_TOP, step, 0)

    sel = vals_s[...] == NEG
    attm = jnp.where(sel, att_all, -3e38)
    mx = attm.max(axis=1, keepdims=True)
    e = jnp.where(sel, jnp.exp(attm - mx), 0.0)
    alpha = e / e.sum(axis=1, keepdims=True)
    vt = (alpha[:, :, None] * v3_ref[...]).sum(axis=1)
    vp = lax.dot_general(vt, wv_ref[...], (((1,), (0,)), ((), ())),
                         preferred_element_type=jnp.float32)
    h = hid_ref[...]
    beta = jax.nn.sigmoid(
        lax.dot_general(h, wgate_ref[...], (((1,), (0,)), ((), ())),
                        preferred_element_type=jnp.float32))
    beta = beta * amask_ref[...]
    hnew_out[...] = h + beta * vp


def _tc_mix(s2, att2, v3, hidden, Wv, wgate, amask, interpret=False):
    return pl.pallas_call(
        _d_body,
        out_shape=jax.ShapeDtypeStruct((NQ, D_MODEL), jnp.float32),
        scratch_shapes=[
            pltpu.VMEM((NQ, NCAND), jnp.float32),
        ],
        interpret=interpret,
    )(s2, att2, v3, hidden, Wv, wgate, amask)


def _e_body(h_ref, emb_ref, logits_out):
    logits_out[...] = lax.dot_general(
        h_ref[...], emb_ref[...], (((1,), (1,)), ((), ())),
        preferred_element_type=jnp.float32)


def _tc_lm_head(hnew, embed_table, interpret=False):
    nvstep = VOCAB // VCHUNK
    return pl.pallas_call(
        _e_body,
        grid=(nvstep,),
        in_specs=[
            pl.BlockSpec((NQ, D_MODEL), lambda i: (0, 0)),
            pl.BlockSpec((VCHUNK, D_MODEL), lambda i: (i, 0)),
        ],
        out_specs=pl.BlockSpec((NQ, VCHUNK), lambda i: (0, i)),
        out_shape=jax.ShapeDtypeStruct((NQ, VOCAB), jnp.float32),
        compiler_params=pltpu.CompilerParams(
            dimension_semantics=("arbitrary",)),
        interpret=interpret,
    )(hnew, embed_table)


def kernel(input_ids, attention_mask, question_time, embed_table, Wq, Wv,
           kb_keys, kb_values, kb_ctx, tau_min, tau_max, w_ctx, w_gate):
    ids = input_ids.reshape(NQ).astype(jnp.int32)

    hidden = _sc_gather_embed(embed_table, ids)

    q, bmax_t = _tc_scores_bmax(hidden, Wq, kb_keys)

    bids = _tc_select_blocks(bmax_t.T)
    bids_flat = bids.reshape(NQ * K_TOP)

    taupack = jnp.concatenate(
        [tau_min.reshape(KB_N // GSIZE, GSIZE),
         tau_max.reshape(KB_N // GSIZE, GSIZE)], axis=1)
    qmin = jnp.repeat(question_time[:, 0], TT).reshape(NQ, 1)
    qmax = jnp.repeat(question_time[:, 1], TT).reshape(NQ, 1)
    s_all, att_all, v_all = _tc_fetch_score(
        bids_flat, kb_keys, kb_values, kb_ctx, taupack,
        q, qmin, qmax, w_ctx.reshape(1, D_CTX))

    amask = attention_mask.reshape(NQ, 1)
    hnew = _tc_mix(
        s_all.reshape(NQ, NCAND), att_all.reshape(NQ, NCAND),
        v_all.reshape(NQ, NCAND, D_V),
        hidden, Wv, w_gate.reshape(D_MODEL, 1), amask)

    logits = _tc_lm_head(hnew, embed_table)
    return logits.reshape(BB, TT, VOCAB)
